# baseline (speedup 1.0000x reference)
import functools

import jax
import jax.numpy as jnp
from jax import lax
from jax.experimental import pallas as pl
from jax.experimental.pallas import tpu as pltpu
from jax.experimental.pallas import tpu_sc as plsc

_ALPHA = 4


def _make_sc_pack(B, T, F, dtype):
    S = T // _ALPHA
    mesh = plsc.VectorSubcoreMesh(core_axis_name="c", subcore_axis_name="s")
    n_workers = 32
    fast_per_w = (B * T) // n_workers

    @functools.partial(
        pl.kernel,
        out_type=(
            jax.ShapeDtypeStruct((B * S, F), dtype),
            jax.ShapeDtypeStruct((B * T, F), dtype),
        ),
        mesh=mesh,
        scratch_types=[pltpu.SemaphoreType.DMA],
    )
    def k(frames_hbm, slow_hbm, fast_hbm, sem):
        wid = lax.axis_index("s") * 2 + lax.axis_index("c")
        b = wid // S
        j = wid % S
        src = b * T + (j * (T - 1)) // (S - 1)
        cp = pltpu.make_async_copy(frames_hbm.at[src], slow_hbm.at[wid], sem)
        cp.start()
        fast_cps = []
        for i in range(fast_per_w):
            r = wid * fast_per_w + i
            c = pltpu.make_async_copy(frames_hbm.at[r], fast_hbm.at[r], sem)
            c.start()
            fast_cps.append(c)
        cp.wait()
        for c in fast_cps:
            c.wait()

    return k


def kernel(frames):
    B, T, C, H, W = frames.shape
    S = T // _ALPHA
    F = C * H * W
    flat = frames.reshape(B * T, F)
    slow, fast = _make_sc_pack(B, T, F, frames.dtype)(flat)
    return slow.reshape(B, S, C, H, W), fast.reshape(B, T, C, H, W)


# speedup vs baseline: 10.4982x; 10.4982x over previous
import functools

import jax
import jax.numpy as jnp
from jax import lax
from jax.experimental import pallas as pl
from jax.experimental.pallas import tpu as pltpu
from jax.experimental.pallas import tpu_sc as plsc

_ALPHA = 4
_SPLIT = 4
_NBUF = 3


def _make_sc_pack(B, T, F, dtype):
    S = T // _ALPHA
    CH = F // _SPLIT
    n_workers = 32
    fast_rows_w = (B * T * _SPLIT) // n_workers
    slow_rows_w = (B * S * _SPLIT) // n_workers
    n_jobs = slow_rows_w + fast_rows_w
    mesh = plsc.VectorSubcoreMesh(core_axis_name="c", subcore_axis_name="s")

    @functools.partial(
        pl.kernel,
        out_type=(
            jax.ShapeDtypeStruct((B * S * F,), dtype),
            jax.ShapeDtypeStruct((B * T * F,), dtype),
        ),
        mesh=mesh,
        scratch_types=[pltpu.VMEM((_NBUF * CH,), dtype)]
        + [pltpu.SemaphoreType.DMA] * (2 * _NBUF),
    )
    def k(src_hbm, slow_hbm, fast_hbm, buf, *sems):
        isem, osem = sems[:_NBUF], sems[_NBUF:]
        wid = lax.axis_index("s") * 2 + lax.axis_index("c")
        b = wid // S
        j = wid % S
        t_src = (j * (T - 1)) // (S - 1)

        def job(i):
            if i < slow_rows_w:
                return ((b * T + t_src) * _SPLIT + i, slow_hbm,
                        wid * slow_rows_w + i)
            r = wid * fast_rows_w + (i - slow_rows_w)
            return r, fast_hbm, r

        ins, outs = [None] * n_jobs, [None] * n_jobs

        def start_in(i):
            src_row, _, _ = job(i)
            ins[i] = pltpu.make_async_copy(
                src_hbm.at[pl.ds(src_row * CH, CH)],
                buf.at[pl.ds((i % _NBUF) * CH, CH)], isem[i % _NBUF])
            ins[i].start()

        def start_out(i):
            _, dref, drow = job(i)
            outs[i] = pltpu.make_async_copy(
                buf.at[pl.ds((i % _NBUF) * CH, CH)],
                dref.at[pl.ds(drow * CH, CH)], osem[i % _NBUF])
            outs[i].start()

        start_in(0)
        for i in range(n_jobs):
            if i + 1 < n_jobs:
                if i + 1 >= _NBUF:
                    outs[i + 1 - _NBUF].wait()
                start_in(i + 1)
            ins[i].wait()
            start_out(i)
        for i in range(n_jobs - _NBUF, n_jobs):
            outs[i].wait()

    return k


def kernel(frames):
    B, T, C, H, W = frames.shape
    S = T // _ALPHA
    F = C * H * W
    flat = frames.reshape(B * T * F)
    slow, fast = _make_sc_pack(B, T, F, frames.dtype)(flat)
    return slow.reshape(B, S, C, H, W), fast.reshape(B, T, C, H, W)


# speedup vs baseline: 11.5856x; 1.1036x over previous
import jax
import jax.numpy as jnp
from jax.experimental import pallas as pl

_ALPHA = 4
_LANES = 128
_GROUP = 8


def kernel(frames):
    B, T, C, H, W = frames.shape
    S = T // _ALPHA
    F = C * H * W
    sub = F // _LANES
    sel_per = _GROUP // _ALPHA
    n_steps = (B * T) // _GROUP

    def body(in_ref, slow_ref, fast_ref):
        fast_ref[...] = in_ref[...]
        i = pl.program_id(0)
        g0 = (i % (T // _GROUP)) * (_GROUP // _ALPHA)
        for k in range(sel_per):
            j = g0 + k
            toff = (j * (T - 1)) // (S - 1) - (i % (T // _GROUP)) * _GROUP
            slow_ref[k] = in_ref[pl.ds(toff, 1)][0]

    flat = frames.reshape(B * T, sub, _LANES)
    slow, fast = pl.pallas_call(
        body,
        grid=(n_steps,),
        in_specs=[pl.BlockSpec((_GROUP, sub, _LANES), lambda i: (i, 0, 0))],
        out_specs=[
            pl.BlockSpec((sel_per, sub, _LANES), lambda i: (i, 0, 0)),
            pl.BlockSpec((_GROUP, sub, _LANES), lambda i: (i, 0, 0)),
        ],
        out_shape=[
            jax.ShapeDtypeStruct((B * S, sub, _LANES), frames.dtype),
            jax.ShapeDtypeStruct((B * T, sub, _LANES), frames.dtype),
        ],
    )(flat)
    return slow.reshape(B, S, C, H, W), fast.reshape(B, T, C, H, W)
